# Initial kernel scaffold; baseline (speedup 1.0000x reference)
import functools

import jax
import jax.numpy as jnp
from jax import lax
from jax.experimental import pallas as pl
from jax.experimental.pallas import tpu as pltpu
from jax.experimental.pallas import tpu_sc as plsc

N_NODES = 10000
N_EDGES = 320000
D = 128

_INFO = plsc.get_sparse_core_info()
NC = _INFO.num_cores
NS = _INFO.num_subcores
NW = NC * NS

CHUNK = 128
EPW = 10240
E_PAD = EPW * NW
STEPS = EPW // CHUNK
N_ACC = 10240
RPS = N_ACC // NS


def _sc_segment_sum(x, src, dst, z2, z1):
    mesh = plsc.VectorSubcoreMesh(core_axis_name="c", subcore_axis_name="s")

    @functools.partial(
        pl.kernel,
        mesh=mesh,
        out_type=(
            jax.ShapeDtypeStruct((NC, N_ACC, D), jnp.float32),
            jax.ShapeDtypeStruct((NC, N_ACC), jnp.float32),
        ),
        scratch_types=[
            pltpu.VMEM((CHUNK,), jnp.int32),
            pltpu.VMEM((CHUNK,), jnp.int32),
            pltpu.VMEM((CHUNK, D), jnp.float32),
            pltpu.VMEM((CHUNK,), jnp.float32),
            pltpu.VMEM_SHARED((N_ACC, D), jnp.float32),
            pltpu.VMEM_SHARED((N_ACC,), jnp.float32),
            pltpu.SemaphoreType.DMA,
        ],
    )
    def k(x_hbm, src_hbm, dst_hbm, z2_hbm, z1_hbm, acc_out, cnt_out,
          src_v, dst_v, rows_v, ones_v, acc_sh, cnt_sh, sem):
        c = lax.axis_index("c")
        s = lax.axis_index("s")
        wid = s * NC + c

        pltpu.sync_copy(z2_hbm.at[pl.ds(s * RPS, RPS)],
                        acc_sh.at[pl.ds(s * RPS, RPS)])
        pltpu.sync_copy(z1_hbm.at[pl.ds(s * RPS, RPS)],
                        cnt_sh.at[pl.ds(s * RPS, RPS)])
        for i in range(CHUNK // 16):
            ones_v[pl.ds(i * 16, 16)] = jnp.ones((16,), jnp.float32)
        plsc.subcore_barrier()

        base = wid * EPW

        def body(j, carry):
            off = base + j * CHUNK
            pltpu.sync_copy(src_hbm.at[pl.ds(off, CHUNK)], src_v)
            pltpu.sync_copy(dst_hbm.at[pl.ds(off, CHUNK)], dst_v)
            pltpu.async_copy(x_hbm.at[dst_v], rows_v, sem).wait()
            pltpu.sync_copy(rows_v, acc_sh.at[src_v], add=True)
            pltpu.sync_copy(ones_v, cnt_sh.at[src_v], add=True)
            return carry

        lax.fori_loop(0, STEPS, body, 0)
        plsc.subcore_barrier()

        pltpu.sync_copy(acc_sh.at[pl.ds(s * RPS, RPS)],
                        acc_out.at[c, pl.ds(s * RPS, RPS)])
        pltpu.sync_copy(cnt_sh.at[pl.ds(s * RPS, RPS)],
                        cnt_out.at[c, pl.ds(s * RPS, RPS)])

    return k(x, src, dst, z2, z1)


_BLK = 1000


def _tc_combine_body(x_ref, acc_ref, cnt_ref, wlt_ref, wrt_ref, b_ref, out_ref):
    i = pl.program_id(0)
    xs = x_ref[...]
    a = acc_ref[0, pl.ds(i * _BLK, _BLK), :] + acc_ref[1, pl.ds(i * _BLK, _BLK), :]
    cn = cnt_ref[0, pl.ds(i * _BLK, _BLK)] + cnt_ref[1, pl.ds(i * _BLK, _BLK)]
    agg = a * (1.0 / jnp.maximum(cn, 1.0))[:, None]
    h = (jnp.dot(xs, wlt_ref[...], preferred_element_type=jnp.float32)
         + jnp.dot(agg, wrt_ref[...], preferred_element_type=jnp.float32)
         + b_ref[...])
    h = jnp.maximum(h, 0.0)
    nrm = jnp.sqrt(jnp.sum(h * h, axis=1, keepdims=True)) + 1e-6
    out_ref[...] = h / nrm


def _tc_combine(x, acc, cnt, wlt, wrt, b):
    grid = N_NODES // _BLK
    return pl.pallas_call(
        _tc_combine_body,
        grid=(grid,),
        in_specs=[
            pl.BlockSpec((_BLK, D), lambda i: (i, 0)),
            pl.BlockSpec((NC, N_ACC, D), lambda i: (0, 0, 0)),
            pl.BlockSpec((NC, N_ACC), lambda i: (0, 0)),
            pl.BlockSpec((D, D), lambda i: (0, 0)),
            pl.BlockSpec((D, D), lambda i: (0, 0)),
            pl.BlockSpec((1, D), lambda i: (0, 0)),
        ],
        out_specs=pl.BlockSpec((_BLK, D), lambda i: (i, 0)),
        out_shape=jax.ShapeDtypeStruct((N_NODES, D), jnp.float32),
    )(x, acc, cnt, wlt, wrt, b)


def kernel(x, edge_index, Wl, bl, Wr, br):
    src = edge_index[0].astype(jnp.int32)
    dst = edge_index[1].astype(jnp.int32)
    pad = E_PAD - N_EDGES
    src_p = jnp.concatenate([src, jnp.full((pad,), N_NODES, jnp.int32)])
    dst_p = jnp.concatenate([dst, jnp.zeros((pad,), jnp.int32)])
    z2 = jnp.zeros((N_ACC, D), jnp.float32)
    z1 = jnp.zeros((N_ACC,), jnp.float32)
    acc, cnt = _sc_segment_sum(x, src_p, dst_p, z2, z1)
    wlt = Wl.T
    wrt = Wr.T
    b = (bl + br).reshape(1, D)
    return _tc_combine(x, acc, cnt, wlt, wrt, b)


# speedup vs baseline: 3.6193x; 3.6193x over previous
import functools

import jax
import jax.numpy as jnp
from jax import lax
from jax.experimental import pallas as pl
from jax.experimental.pallas import tpu as pltpu
from jax.experimental.pallas import tpu_sc as plsc

N_NODES = 10000
N_EDGES = 320000
D = 128

_INFO = plsc.get_sparse_core_info()
NC = _INFO.num_cores
NS = _INFO.num_subcores
NW = NC * NS

CHUNK = 128
EPW = 10240
E_PAD = EPW * NW
STEPS = EPW // CHUNK
N_ACC = 10240
RPS = N_ACC // NS


def _sc_segment_sum(x, src, dst, z2, z1):
    mesh = plsc.VectorSubcoreMesh(core_axis_name="c", subcore_axis_name="s")

    @functools.partial(
        pl.kernel,
        mesh=mesh,
        out_type=(
            jax.ShapeDtypeStruct((NC, N_ACC, D), jnp.float32),
            jax.ShapeDtypeStruct((NC, N_ACC), jnp.float32),
        ),
        scratch_types=[
            pltpu.VMEM((CHUNK,), jnp.int32),
            pltpu.VMEM((CHUNK,), jnp.int32),
            pltpu.VMEM((CHUNK, D), jnp.float32),
            pltpu.VMEM((CHUNK,), jnp.float32),
            pltpu.VMEM_SHARED((N_ACC, D), jnp.float32),
            pltpu.VMEM_SHARED((N_ACC,), jnp.float32),
            pltpu.SemaphoreType.DMA,
        ],
    )
    def k(x_hbm, src_hbm, dst_hbm, z2_hbm, z1_hbm, acc_out, cnt_out,
          src_v, dst_v, rows_v, ones_v, acc_sh, cnt_sh, sem):
        c = lax.axis_index("c")
        s = lax.axis_index("s")
        wid = s * NC + c

        pltpu.sync_copy(z2_hbm.at[pl.ds(s * RPS, RPS)],
                        acc_sh.at[pl.ds(s * RPS, RPS)])
        pltpu.sync_copy(z1_hbm.at[pl.ds(s * RPS, RPS)],
                        cnt_sh.at[pl.ds(s * RPS, RPS)])
        for i in range(CHUNK // 16):
            ones_v[pl.ds(i * 16, 16)] = jnp.ones((16,), jnp.float32)
        plsc.subcore_barrier()

        base = wid * EPW

        def body(j, carry):
            off = base + j * CHUNK
            pltpu.sync_copy(src_hbm.at[pl.ds(off, CHUNK)], src_v)
            pltpu.sync_copy(dst_hbm.at[pl.ds(off, CHUNK)], dst_v)
            pltpu.async_copy(x_hbm.at[dst_v], rows_v, sem).wait()
            pltpu.sync_copy(rows_v, acc_sh.at[src_v], add=True)
            pltpu.sync_copy(ones_v, cnt_sh.at[src_v], add=True)
            return carry

        lax.fori_loop(0, STEPS, body, 0)
        plsc.subcore_barrier()

        pltpu.sync_copy(acc_sh.at[pl.ds(s * RPS, RPS)],
                        acc_out.at[c, pl.ds(s * RPS, RPS)])
        pltpu.sync_copy(cnt_sh.at[pl.ds(s * RPS, RPS)],
                        cnt_out.at[c, pl.ds(s * RPS, RPS)])

    return k(x, src, dst, z2, z1)


_BLK = 2048


def _tc_combine_body(x_ref, acc_ref, cnt_ref, wlt_ref, wrt_ref, b_ref, out_ref):
    xs = x_ref[...]
    a = acc_ref[0] + acc_ref[1]
    cn = cnt_ref[0] + cnt_ref[1]
    agg = a * (1.0 / jnp.maximum(cn, 1.0))[:, None]
    h = (jnp.dot(xs, wlt_ref[...], preferred_element_type=jnp.float32)
         + jnp.dot(agg, wrt_ref[...], preferred_element_type=jnp.float32)
         + b_ref[...])
    h = jnp.maximum(h, 0.0)
    nrm = jnp.sqrt(jnp.sum(h * h, axis=1, keepdims=True)) + 1e-6
    out_ref[...] = h / nrm


def _tc_combine(x, acc, cnt, wlt, wrt, b):
    grid = N_ACC // _BLK
    return pl.pallas_call(
        _tc_combine_body,
        grid=(grid,),
        in_specs=[
            pl.BlockSpec((_BLK, D), lambda i: (i, 0)),
            pl.BlockSpec((NC, _BLK, D), lambda i: (0, i, 0)),
            pl.BlockSpec((NC, _BLK), lambda i: (0, i)),
            pl.BlockSpec((D, D), lambda i: (0, 0)),
            pl.BlockSpec((D, D), lambda i: (0, 0)),
            pl.BlockSpec((1, D), lambda i: (0, 0)),
        ],
        out_specs=pl.BlockSpec((_BLK, D), lambda i: (i, 0)),
        out_shape=jax.ShapeDtypeStruct((N_NODES, D), jnp.float32),
    )(x, acc, cnt, wlt, wrt, b)


def kernel(x, edge_index, Wl, bl, Wr, br):
    src = edge_index[0].astype(jnp.int32)
    dst = edge_index[1].astype(jnp.int32)
    pad = E_PAD - N_EDGES
    src_p = jnp.concatenate([src, jnp.full((pad,), N_NODES, jnp.int32)])
    dst_p = jnp.concatenate([dst, jnp.zeros((pad,), jnp.int32)])
    z2 = jnp.zeros((N_ACC, D), jnp.float32)
    z1 = jnp.zeros((N_ACC,), jnp.float32)
    acc, cnt = _sc_segment_sum(x, src_p, dst_p, z2, z1)
    wlt = Wl.T
    wrt = Wr.T
    b = (bl + br).reshape(1, D)
    return _tc_combine(x, acc, cnt, wlt, wrt, b)


# speedup vs baseline: 5.0914x; 1.4067x over previous
import functools

import jax
import jax.numpy as jnp
from jax import lax
from jax.experimental import pallas as pl
from jax.experimental.pallas import tpu as pltpu
from jax.experimental.pallas import tpu_sc as plsc

N_NODES = 10000
N_EDGES = 320000
D = 128

_INFO = plsc.get_sparse_core_info()
NC = _INFO.num_cores
NS = _INFO.num_subcores
NW = NC * NS

CHUNK = 128
EPW = N_EDGES // NW
STEPS = 80
N_ACC = 10240
RPS = N_ACC // NS
SHIFT = 14
MASK = (1 << SHIFT) - 1


def _sc_segment_sum(x, packed, z2, z1):
    mesh = plsc.VectorSubcoreMesh(core_axis_name="c", subcore_axis_name="s")

    @functools.partial(
        pl.kernel,
        mesh=mesh,
        out_type=(
            jax.ShapeDtypeStruct((NC, N_ACC, D), jnp.float32),
            jax.ShapeDtypeStruct((NC, N_ACC), jnp.float32),
        ),
        scratch_types=[
            pltpu.VMEM((STEPS, CHUNK), jnp.int32),
            pltpu.VMEM((CHUNK,), jnp.int32),
            pltpu.VMEM((CHUNK,), jnp.int32),
            pltpu.VMEM((CHUNK,), jnp.int32),
            pltpu.VMEM((CHUNK,), jnp.int32),
            pltpu.VMEM((CHUNK, D), jnp.float32),
            pltpu.VMEM((CHUNK, D), jnp.float32),
            pltpu.VMEM((CHUNK,), jnp.float32),
            pltpu.VMEM_SHARED((N_ACC, D), jnp.float32),
            pltpu.VMEM_SHARED((N_ACC,), jnp.float32),
            pltpu.SemaphoreType.DMA,
            pltpu.SemaphoreType.DMA,
        ],
    )
    def k(x_hbm, pk_hbm, z2_hbm, z1_hbm, acc_out, cnt_out,
          pk_v, src0, src1, dst0, dst1, rows0, rows1, ones_v, acc_sh, cnt_sh,
          sem0, sem1):
        c = lax.axis_index("c")
        s = lax.axis_index("s")
        wid = s * NC + c

        pltpu.sync_copy(z2_hbm.at[pl.ds(s * RPS, RPS)],
                        acc_sh.at[pl.ds(s * RPS, RPS)])
        pltpu.sync_copy(z1_hbm.at[pl.ds(s * RPS, RPS)],
                        cnt_sh.at[pl.ds(s * RPS, RPS)])
        pltpu.sync_copy(pk_hbm.at[pl.ds(wid * STEPS, STEPS)], pk_v)
        for i in range(CHUNK // 16):
            ones_v[pl.ds(i * 16, 16)] = jnp.ones((16,), jnp.float32)
        plsc.subcore_barrier()

        srcs = (src0, src1)
        dsts = (dst0, dst1)
        bufs = (rows0, rows1)
        sems = (sem0, sem1)

        def unpack(r, src_c, dst_c):
            for kk in range(CHUNK // 16):
                col = kk * 16
                v = pk_v[r, pl.ds(col, 16)]
                src_c[pl.ds(col, 16)] = lax.shift_right_logical(v, SHIFT)
                dst_c[pl.ds(col, 16)] = lax.bitwise_and(v, MASK)

        for b in range(2):
            unpack(b, srcs[b], dsts[b])
            pltpu.async_copy(x_hbm.at[dsts[b]], bufs[b], sems[b])

        def body(i, carry):
            j = 2 * i
            for b in range(2):
                jb = j + b
                buf, sem = bufs[b], sems[b]
                pltpu.make_async_copy(x_hbm.at[dsts[b]], buf, sem).wait()
                pltpu.sync_copy(buf, acc_sh.at[srcs[b]], add=True)
                pltpu.sync_copy(ones_v, cnt_sh.at[srcs[b]], add=True)
                nxt = lax.rem(jb + 2, STEPS)
                unpack(nxt, srcs[b], dsts[b])
                pltpu.async_copy(x_hbm.at[dsts[b]], buf, sem)
            return carry

        lax.fori_loop(0, STEPS // 2, body, 0)
        pltpu.make_async_copy(x_hbm.at[dst0], rows0, sem0).wait()
        pltpu.make_async_copy(x_hbm.at[dst1], rows1, sem1).wait()
        plsc.subcore_barrier()

        pltpu.sync_copy(acc_sh.at[pl.ds(s * RPS, RPS)],
                        acc_out.at[c, pl.ds(s * RPS, RPS)])
        pltpu.sync_copy(cnt_sh.at[pl.ds(s * RPS, RPS)],
                        cnt_out.at[c, pl.ds(s * RPS, RPS)])

    return k(x, packed, z2, z1)


_BLK = 2048


def _tc_combine_body(x_ref, acc_ref, cnt_ref, wlt_ref, wrt_ref, b_ref, out_ref):
    xs = x_ref[...]
    a = acc_ref[0] + acc_ref[1]
    cn = cnt_ref[0] + cnt_ref[1]
    agg = a * (1.0 / jnp.maximum(cn, 1.0))[:, None]
    h = (jnp.dot(xs, wlt_ref[...], preferred_element_type=jnp.float32)
         + jnp.dot(agg, wrt_ref[...], preferred_element_type=jnp.float32)
         + b_ref[...])
    h = jnp.maximum(h, 0.0)
    nrm = jnp.sqrt(jnp.sum(h * h, axis=1, keepdims=True)) + 1e-6
    out_ref[...] = h / nrm


def _tc_combine(x, acc, cnt, wlt, wrt, b):
    grid = N_ACC // _BLK
    return pl.pallas_call(
        _tc_combine_body,
        grid=(grid,),
        in_specs=[
            pl.BlockSpec((_BLK, D), lambda i: (i, 0)),
            pl.BlockSpec((NC, _BLK, D), lambda i: (0, i, 0)),
            pl.BlockSpec((NC, _BLK), lambda i: (0, i)),
            pl.BlockSpec((D, D), lambda i: (0, 0)),
            pl.BlockSpec((D, D), lambda i: (0, 0)),
            pl.BlockSpec((1, D), lambda i: (0, 0)),
        ],
        out_specs=pl.BlockSpec((_BLK, D), lambda i: (i, 0)),
        out_shape=jax.ShapeDtypeStruct((N_NODES, D), jnp.float32),
    )(x, acc, cnt, wlt, wrt, b)


def kernel(x, edge_index, Wl, bl, Wr, br):
    src = edge_index[0].astype(jnp.int32).reshape(NW, EPW)
    dst = edge_index[1].astype(jnp.int32).reshape(NW, EPW)
    pad = STEPS * CHUNK - EPW
    src_p = jnp.pad(src, ((0, 0), (0, pad)), constant_values=N_NODES)
    dst_p = jnp.pad(dst, ((0, 0), (0, pad)))
    packed = (jnp.left_shift(src_p, SHIFT) | dst_p).reshape(NW * STEPS, CHUNK)
    z2 = jnp.zeros((N_ACC, D), jnp.float32)
    z1 = jnp.zeros((N_ACC,), jnp.float32)
    acc, cnt = _sc_segment_sum(x, packed, z2, z1)
    wlt = Wl.T
    wrt = Wr.T
    b = (bl + br).reshape(1, D)
    return _tc_combine(x, acc, cnt, wlt, wrt, b)


# speedup vs baseline: 5.1096x; 1.0036x over previous
import functools

import jax
import jax.numpy as jnp
from jax import lax
from jax.experimental import pallas as pl
from jax.experimental.pallas import tpu as pltpu
from jax.experimental.pallas import tpu_sc as plsc

N_NODES = 10000
N_EDGES = 320000
D = 128

_INFO = plsc.get_sparse_core_info()
NC = _INFO.num_cores
NS = _INFO.num_subcores
NW = NC * NS

CHUNK = 128
EPW = N_EDGES // NW
STEPS = 80
N_ACC = 10240
RPS = N_ACC // NS
SHIFT = 14
MASK = (1 << SHIFT) - 1


def _sc_segment_sum(x, packed, z2, z1):
    mesh = plsc.VectorSubcoreMesh(core_axis_name="c", subcore_axis_name="s")

    @functools.partial(
        pl.kernel,
        mesh=mesh,
        out_type=(
            jax.ShapeDtypeStruct((NC, N_ACC, D), jnp.float32),
            jax.ShapeDtypeStruct((NC, N_ACC), jnp.float32),
        ),
        scratch_types=[
            pltpu.VMEM((STEPS, CHUNK), jnp.int32),
            pltpu.VMEM((CHUNK,), jnp.int32),
            pltpu.VMEM((CHUNK,), jnp.int32),
            pltpu.VMEM((CHUNK,), jnp.int32),
            pltpu.VMEM((CHUNK,), jnp.int32),
            pltpu.VMEM((CHUNK, D), jnp.float32),
            pltpu.VMEM((CHUNK, D), jnp.float32),
            pltpu.VMEM((CHUNK,), jnp.float32),
            pltpu.VMEM_SHARED((N_ACC, D), jnp.float32),
            pltpu.VMEM_SHARED((N_ACC,), jnp.float32),
            pltpu.SemaphoreType.DMA,
            pltpu.SemaphoreType.DMA,
            pltpu.SemaphoreType.DMA,
            pltpu.SemaphoreType.DMA,
        ],
    )
    def k(x_hbm, pk_hbm, z2_hbm, z1_hbm, acc_out, cnt_out,
          pk_v, src0, src1, dst0, dst1, rows0, rows1, ones_v, acc_sh, cnt_sh,
          sem0, sem1, csem0, csem1):
        c = lax.axis_index("c")
        s = lax.axis_index("s")
        wid = s * NC + c

        pltpu.sync_copy(z2_hbm.at[pl.ds(s * RPS, RPS)],
                        acc_sh.at[pl.ds(s * RPS, RPS)])
        pltpu.sync_copy(z1_hbm.at[pl.ds(s * RPS, RPS)],
                        cnt_sh.at[pl.ds(s * RPS, RPS)])
        pltpu.sync_copy(pk_hbm.at[pl.ds(wid * STEPS, STEPS)], pk_v)
        for i in range(CHUNK // 16):
            ones_v[pl.ds(i * 16, 16)] = jnp.ones((16,), jnp.float32)
        plsc.subcore_barrier()

        srcs = (src0, src1)
        dsts = (dst0, dst1)
        bufs = (rows0, rows1)
        sems = (sem0, sem1)

        def unpack(r, src_c, dst_c):
            for kk in range(CHUNK // 16):
                col = kk * 16
                v = pk_v[r, pl.ds(col, 16)]
                src_c[pl.ds(col, 16)] = lax.shift_right_logical(v, SHIFT)
                dst_c[pl.ds(col, 16)] = lax.bitwise_and(v, MASK)

        for b in range(2):
            unpack(b, srcs[b], dsts[b])
            pltpu.async_copy(x_hbm.at[dsts[b]], bufs[b], sems[b])

        csems = (csem0, csem1)

        def body(i, carry):
            j = 2 * i
            for b in range(2):
                jb = j + b
                buf, sem, csem = bufs[b], sems[b], csems[b]
                pltpu.make_async_copy(x_hbm.at[dsts[b]], buf, sem).wait()
                pltpu.async_copy(ones_v, cnt_sh.at[srcs[b]], csem, add=True)
                pltpu.sync_copy(buf, acc_sh.at[srcs[b]], add=True)
                pltpu.make_async_copy(ones_v, cnt_sh.at[srcs[b]], csem).wait()
                nxt = lax.rem(jb + 2, STEPS)
                unpack(nxt, srcs[b], dsts[b])
                pltpu.async_copy(x_hbm.at[dsts[b]], buf, sem)
            return carry

        lax.fori_loop(0, STEPS // 2, body, 0)
        pltpu.make_async_copy(x_hbm.at[dst0], rows0, sem0).wait()
        pltpu.make_async_copy(x_hbm.at[dst1], rows1, sem1).wait()
        plsc.subcore_barrier()

        pltpu.sync_copy(acc_sh.at[pl.ds(s * RPS, RPS)],
                        acc_out.at[c, pl.ds(s * RPS, RPS)])
        pltpu.sync_copy(cnt_sh.at[pl.ds(s * RPS, RPS)],
                        cnt_out.at[c, pl.ds(s * RPS, RPS)])

    return k(x, packed, z2, z1)


_BLK = 2048


def _tc_combine_body(x_ref, acc_ref, cnt_ref, wlt_ref, wrt_ref, b_ref, out_ref):
    xs = x_ref[...]
    a = acc_ref[0] + acc_ref[1]
    cn = cnt_ref[0] + cnt_ref[1]
    agg = a * (1.0 / jnp.maximum(cn, 1.0))[:, None]
    h = (jnp.dot(xs, wlt_ref[...], preferred_element_type=jnp.float32)
         + jnp.dot(agg, wrt_ref[...], preferred_element_type=jnp.float32)
         + b_ref[...])
    h = jnp.maximum(h, 0.0)
    nrm = jnp.sqrt(jnp.sum(h * h, axis=1, keepdims=True)) + 1e-6
    out_ref[...] = h / nrm


def _tc_combine(x, acc, cnt, wlt, wrt, b):
    grid = N_ACC // _BLK
    return pl.pallas_call(
        _tc_combine_body,
        grid=(grid,),
        in_specs=[
            pl.BlockSpec((_BLK, D), lambda i: (i, 0)),
            pl.BlockSpec((NC, _BLK, D), lambda i: (0, i, 0)),
            pl.BlockSpec((NC, _BLK), lambda i: (0, i)),
            pl.BlockSpec((D, D), lambda i: (0, 0)),
            pl.BlockSpec((D, D), lambda i: (0, 0)),
            pl.BlockSpec((1, D), lambda i: (0, 0)),
        ],
        out_specs=pl.BlockSpec((_BLK, D), lambda i: (i, 0)),
        out_shape=jax.ShapeDtypeStruct((N_NODES, D), jnp.float32),
    )(x, acc, cnt, wlt, wrt, b)


def kernel(x, edge_index, Wl, bl, Wr, br):
    src = edge_index[0].astype(jnp.int32).reshape(NW, EPW)
    dst = edge_index[1].astype(jnp.int32).reshape(NW, EPW)
    pad = STEPS * CHUNK - EPW
    src_p = jnp.pad(src, ((0, 0), (0, pad)), constant_values=N_NODES)
    dst_p = jnp.pad(dst, ((0, 0), (0, pad)))
    packed = (jnp.left_shift(src_p, SHIFT) | dst_p).reshape(NW * STEPS, CHUNK)
    z2 = jnp.zeros((N_ACC, D), jnp.float32)
    z1 = jnp.zeros((N_ACC,), jnp.float32)
    acc, cnt = _sc_segment_sum(x, packed, z2, z1)
    wlt = Wl.T
    wrt = Wr.T
    b = (bl + br).reshape(1, D)
    return _tc_combine(x, acc, cnt, wlt, wrt, b)
